# baseline (speedup 1.0000x reference)
import jax
import jax.numpy as jnp
import numpy as np
from jax import lax
from jax.experimental import pallas as pl
from jax.experimental.pallas import tpu as pltpu
from jax.experimental.pallas import tpu_sc as plsc

N_NODES = 10000
D = 128
NC, NS = 2, 16
NW = NC * NS
C = 128
NCHUNK = 80
ROUNDS = 5
CH = NCHUNK // ROUNDS
E_PAD = NW * NCHUNK * C
N_PAD = 10240
TRASH = N_NODES
ROWS_PER_SUB = N_PAD // NS
RCHUNKS = ROWS_PER_SUB // C
HROWS = N_PAD // C
FBASE = NC * N_PAD


def _sc_body(xt, sd, outf, src_v, dst_v, buf0, buf1, hist, acc,
             sem0, sem1, ssem0, ssem1):
  c = lax.axis_index("c")
  s = lax.axis_index("s")
  wid = c * NS + s
  base = s * ROWS_PER_SUB
  z16 = jnp.zeros((16,), jnp.float32)

  @pl.loop(0, C)
  def _(r):
    for k in range(D // 16):
      buf0[r, pl.ds(k * 16, 16)] = z16

  @pl.loop(0, HROWS)
  def _(r):
    for k in range(C // 16):
      hist[r, pl.ds(k * 16, 16)] = z16

  for t in range(RCHUNKS):
    pltpu.sync_copy(buf0, acc.at[pl.ds(base + t * C, C)])

  plsc.subcore_barrier()

  for r in range(ROUNDS):
    pltpu.sync_copy(sd.at[pl.ds(wid * NCHUNK + r * CH, CH)], src_v)
    pltpu.sync_copy(
        sd.at[pl.ds(NW * NCHUNK + wid * NCHUNK + r * CH, CH)], dst_v)

    pltpu.async_copy(xt.at[src_v.at[0]], buf0, sem0)
    pltpu.async_copy(xt.at[src_v.at[1]], buf1, sem1)

    def _hist(j):
      for k in range(C // 16):
        v = dst_v[j, pl.ds(k * 16, 16)]
        cnt, lastm = plsc.scan_count(v)
        plsc.addupdate_scatter(
            hist,
            [lax.shift_right_logical(v, 7), jnp.bitwise_and(v, 127)],
            cnt.astype(jnp.float32), mask=lastm)

    pltpu.make_async_copy(xt.at[src_v.at[0]], buf0, sem0).wait()
    pltpu.async_copy(buf0, acc.at[dst_v.at[0]], ssem0, add=True)
    _hist(0)

    @pl.loop(0, (CH - 2) // 2)
    def _(i):
      for b, (buf, sem, ssem, obuf, osem, ossem) in enumerate((
          (buf1, sem1, ssem1, buf0, sem0, ssem0),
          (buf0, sem0, ssem0, buf1, sem1, ssem1))):
        j = i * 2 + 1 + b
        pltpu.make_async_copy(xt.at[src_v.at[j]], buf, sem).wait()
        pltpu.async_copy(buf, acc.at[dst_v.at[j]], ssem, add=True)
        _hist(j)
        pltpu.make_async_copy(obuf, acc.at[dst_v.at[j - 1]], ossem).wait()
        pltpu.async_copy(xt.at[src_v.at[j + 1]], obuf, osem)

    pltpu.make_async_copy(xt.at[src_v.at[CH - 1]], buf1, sem1).wait()
    pltpu.async_copy(buf1, acc.at[dst_v.at[CH - 1]], ssem1, add=True)
    _hist(CH - 1)
    pltpu.make_async_copy(buf0, acc.at[dst_v.at[CH - 2]], ssem0).wait()
    pltpu.make_async_copy(buf1, acc.at[dst_v.at[CH - 1]], ssem1).wait()

  pltpu.sync_copy(hist, outf.at[pl.ds(FBASE + wid * HROWS, HROWS)])

  plsc.subcore_barrier()

  pltpu.sync_copy(acc.at[pl.ds(base, ROWS_PER_SUB)],
                  outf.at[pl.ds(c * N_PAD + base, ROWS_PER_SUB)])


_sc_scatter_cache = []


def _sc_scatter(*args):
  if not _sc_scatter_cache:
    mesh = plsc.VectorSubcoreMesh(
        core_axis_name="c", subcore_axis_name="s",
        num_cores=NC, num_subcores=NS)
    _sc_scatter_cache.append(pl.kernel(
        _sc_body,
        out_type=jax.ShapeDtypeStruct((FBASE + NW * HROWS, D), jnp.float32),
        mesh=mesh,
        compiler_params=pltpu.CompilerParams(needs_layout_passes=False),
        scratch_types=[
            pltpu.VMEM((CH, C), jnp.int32),
            pltpu.VMEM((CH, C), jnp.int32),
            pltpu.VMEM((C, D), jnp.float32),
            pltpu.VMEM((C, D), jnp.float32),
            pltpu.VMEM((HROWS, C), jnp.float32),
            pltpu.VMEM_SHARED((N_PAD, D), jnp.float32),
            pltpu.SemaphoreType.DMA,
            pltpu.SemaphoreType.DMA,
            pltpu.SemaphoreType.DMA,
            pltpu.SemaphoreType.DMA,
        ],
    ))
  return _sc_scatter_cache[0](*args)


NCHK = NW * NCHUNK
PADCH = NCHK - 320000 // C


def _prep_body(e_ref, pad_ref, o_ref):
  e = e_ref.shape[1]
  src = e_ref[0].reshape(e // C, C)
  dst = e_ref[1].reshape(e // C, C)
  o_ref[...] = jnp.concatenate([src, pad_ref[0], dst, pad_ref[1]], axis=0)


def _prep(ei, pads):
  e = ei.shape[1]
  return pl.pallas_call(
      _prep_body,
      grid=(1,),
      in_specs=[
          pl.BlockSpec((2, e), lambda i: (0, 0)),
          pl.BlockSpec((2, PADCH, C), lambda i: (0, 0, 0)),
      ],
      out_specs=pl.BlockSpec((2 * NCHK, C), lambda i: (0, 0)),
      out_shape=jax.ShapeDtypeStruct((2 * NCHK, C), jnp.int32),
  )(ei, pads)


BN = 1024


def _mlp_body(p0_ref, p1_ref, dp_ref, w1t_ref, b1_ref, w2t_ref, b2_ref,
              o_ref):
  agr = p0_ref[...] + p1_ref[...]
  deg = jnp.sum(dp_ref[...], axis=0)[:, None]
  xn = agr / (deg + 1e-8)
  h = jnp.tanh(
      jnp.dot(xn, w1t_ref[...], preferred_element_type=jnp.float32)
      + b1_ref[...])
  o_ref[...] = (
      jnp.dot(h, w2t_ref[...], preferred_element_type=jnp.float32)
      + b2_ref[...])


def _mlp(outf, dp, w1t, b1, w2t, b2):
  grid = N_PAD // BN
  return pl.pallas_call(
      _mlp_body,
      grid=(grid,),
      in_specs=[
          pl.BlockSpec((BN, D), lambda i: (i, 0)),
          pl.BlockSpec((BN, D), lambda i: (N_PAD // BN + i, 0)),
          pl.BlockSpec((NW, BN), lambda i: (0, i)),
          pl.BlockSpec((D, D), lambda i: (0, 0)),
          pl.BlockSpec((1, D), lambda i: (0, 0)),
          pl.BlockSpec((D, D), lambda i: (0, 0)),
          pl.BlockSpec((1, D), lambda i: (0, 0)),
      ],
      out_specs=pl.BlockSpec((BN, D), lambda i: (i, 0)),
      out_shape=jax.ShapeDtypeStruct((N_PAD, D), jnp.float32),
  )(outf, outf, dp, w1t, b1, w2t, b2)


def kernel(x, edge_index, W1, b1, W2, b2):
  ei = edge_index.astype(jnp.int32)
  pad_i = np.arange(PADCH * C, dtype=np.int32)
  pads = jnp.asarray(np.stack([
      pad_i % N_NODES,
      TRASH + pad_i % (N_PAD - N_NODES),
  ]).reshape(2, PADCH, C))
  sd = _prep(ei, pads)
  outf = _sc_scatter(x, sd)
  dp = outf[FBASE:].reshape(NW, N_PAD)
  out = _mlp(outf, dp, W1.T, b1.reshape(1, D), W2.T, b2.reshape(1, D))
  return out[:N_NODES]


# speedup vs baseline: 1.1312x; 1.1312x over previous
import jax
import jax.numpy as jnp
import numpy as np
from jax import lax
from jax.experimental import pallas as pl
from jax.experimental.pallas import tpu as pltpu
from jax.experimental.pallas import tpu_sc as plsc

N_NODES = 10000
D = 128
NC, NS = 2, 16
NW = NC * NS
C = 128
NCHUNK = 80
ROUNDS = 5
CH = NCHUNK // ROUNDS
E_PAD = NW * NCHUNK * C
N_PAD = 10240
TRASH = N_NODES
ROWS_PER_SUB = N_PAD // NS
RCHUNKS = ROWS_PER_SUB // C
HROWS = N_PAD // C
FBASE = NC * N_PAD


def _sc_body(xt, sd, outf, src_v, dst_v, buf0, buf1, hist, acc,
             sem0, sem1, ssem0, ssem1):
  c = lax.axis_index("c")
  s = lax.axis_index("s")
  wid = c * NS + s
  base = s * ROWS_PER_SUB
  z16 = jnp.zeros((16,), jnp.float32)

  @pl.loop(0, C)
  def _(r):
    for k in range(D // 16):
      buf0[r, pl.ds(k * 16, 16)] = z16

  @pl.loop(0, HROWS)
  def _(r):
    for k in range(C // 16):
      hist[r, pl.ds(k * 16, 16)] = z16

  for t in range(RCHUNKS):
    pltpu.sync_copy(buf0, acc.at[pl.ds(base + t * C, C)])

  plsc.subcore_barrier()

  for r in range(ROUNDS):
    pltpu.sync_copy(sd.at[pl.ds(wid * NCHUNK + r * CH, CH)], src_v)
    pltpu.sync_copy(
        sd.at[pl.ds(NW * NCHUNK + wid * NCHUNK + r * CH, CH)], dst_v)

    pltpu.async_copy(xt.at[src_v.at[0]], buf0, sem0)
    pltpu.async_copy(xt.at[src_v.at[1]], buf1, sem1)

    @pl.loop(0, CH // 2)
    def _(i):
      j0 = i * 2
      for b, (buf, sem, ssem) in enumerate(
          ((buf0, sem0, ssem0), (buf1, sem1, ssem1))):
        j = j0 + b
        pltpu.make_async_copy(xt.at[src_v.at[j]], buf, sem).wait()
        pltpu.async_copy(buf, acc.at[dst_v.at[j]], ssem, add=True)
        for k in range(C // 16):
          v = dst_v[j, pl.ds(k * 16, 16)]
          cnt, lastm = plsc.scan_count(v)
          plsc.addupdate_scatter(
              hist,
              [lax.shift_right_logical(v, 7), jnp.bitwise_and(v, 127)],
              cnt.astype(jnp.float32), mask=lastm)
        pltpu.make_async_copy(buf, acc.at[dst_v.at[j]], ssem).wait()
        jn = jnp.minimum(j + 2, CH - 1)
        pltpu.async_copy(xt.at[src_v.at[jn]], buf, sem)

    pltpu.make_async_copy(xt.at[src_v.at[CH - 1]], buf0, sem0).wait()
    pltpu.make_async_copy(xt.at[src_v.at[CH - 1]], buf1, sem1).wait()

  pltpu.sync_copy(hist, outf.at[pl.ds(FBASE + wid * HROWS, HROWS)])

  plsc.subcore_barrier()

  pltpu.sync_copy(acc.at[pl.ds(base, ROWS_PER_SUB)],
                  outf.at[pl.ds(c * N_PAD + base, ROWS_PER_SUB)])


_sc_scatter_cache = []


def _sc_scatter(*args):
  if not _sc_scatter_cache:
    mesh = plsc.VectorSubcoreMesh(
        core_axis_name="c", subcore_axis_name="s",
        num_cores=NC, num_subcores=NS)
    _sc_scatter_cache.append(pl.kernel(
        _sc_body,
        out_type=jax.ShapeDtypeStruct((FBASE + NW * HROWS, D), jnp.float32),
        mesh=mesh,
        compiler_params=pltpu.CompilerParams(needs_layout_passes=False),
        scratch_types=[
            pltpu.VMEM((CH, C), jnp.int32),
            pltpu.VMEM((CH, C), jnp.int32),
            pltpu.VMEM((C, D), jnp.float32),
            pltpu.VMEM((C, D), jnp.float32),
            pltpu.VMEM((HROWS, C), jnp.float32),
            pltpu.VMEM_SHARED((N_PAD, D), jnp.float32),
            pltpu.SemaphoreType.DMA,
            pltpu.SemaphoreType.DMA,
            pltpu.SemaphoreType.DMA,
            pltpu.SemaphoreType.DMA,
        ],
    ))
  return _sc_scatter_cache[0](*args)


NCHK = NW * NCHUNK
PADCH = NCHK - 320000 // C


def _prep_body(e_ref, pad_ref, o_ref):
  e = e_ref.shape[1]
  src = e_ref[0].reshape(e // C, C)
  dst = e_ref[1].reshape(e // C, C)
  o_ref[...] = jnp.concatenate([src, pad_ref[0], dst, pad_ref[1]], axis=0)


def _prep(ei, pads):
  e = ei.shape[1]
  return pl.pallas_call(
      _prep_body,
      grid=(1,),
      in_specs=[
          pl.BlockSpec((2, e), lambda i: (0, 0)),
          pl.BlockSpec((2, PADCH, C), lambda i: (0, 0, 0)),
      ],
      out_specs=pl.BlockSpec((2 * NCHK, C), lambda i: (0, 0)),
      out_shape=jax.ShapeDtypeStruct((2 * NCHK, C), jnp.int32),
  )(ei, pads)


BN = 1024


def _mlp_body(p0_ref, p1_ref, dp_ref, w1t_ref, b1_ref, w2t_ref, b2_ref,
              o_ref):
  agr = p0_ref[...] + p1_ref[...]
  deg = jnp.sum(dp_ref[...], axis=0)[:, None]
  xn = agr / (deg + 1e-8)
  h = jnp.tanh(
      jnp.dot(xn, w1t_ref[...], preferred_element_type=jnp.float32)
      + b1_ref[...])
  o_ref[...] = (
      jnp.dot(h, w2t_ref[...], preferred_element_type=jnp.float32)
      + b2_ref[...])


def _mlp(outf, dp, w1t, b1, w2t, b2):
  grid = N_PAD // BN
  return pl.pallas_call(
      _mlp_body,
      grid=(grid,),
      in_specs=[
          pl.BlockSpec((BN, D), lambda i: (i, 0)),
          pl.BlockSpec((BN, D), lambda i: (N_PAD // BN + i, 0)),
          pl.BlockSpec((NW, BN), lambda i: (0, i)),
          pl.BlockSpec((D, D), lambda i: (0, 0)),
          pl.BlockSpec((1, D), lambda i: (0, 0)),
          pl.BlockSpec((D, D), lambda i: (0, 0)),
          pl.BlockSpec((1, D), lambda i: (0, 0)),
      ],
      out_specs=pl.BlockSpec((BN, D), lambda i: (i, 0)),
      out_shape=jax.ShapeDtypeStruct((N_PAD, D), jnp.float32),
  )(outf, outf, dp, w1t, b1, w2t, b2)


def kernel(x, edge_index, W1, b1, W2, b2):
  ei = edge_index.astype(jnp.int32)
  pad_i = np.arange(PADCH * C, dtype=np.int32)
  pads = jnp.asarray(np.stack([
      pad_i % N_NODES,
      TRASH + pad_i % (N_PAD - N_NODES),
  ]).reshape(2, PADCH, C))
  sd = _prep(ei, pads)
  outf = _sc_scatter(x, sd)
  dp = outf[FBASE:].reshape(NW, N_PAD)
  out = _mlp(outf, dp, W1.T, b1.reshape(1, D), W2.T, b2.reshape(1, D))
  return out[:N_NODES]
